# Initial kernel scaffold; baseline (speedup 1.0000x reference)
import jax
import jax.numpy as jnp
from jax.experimental import pallas as pl


def kernel(x, edge_index, batch, W_in, b_in, W_ih, W_hh, b_ih, b_hh, W_pred, b_pred):
    raise NotImplementedError("write your pallas kernel here")


# speedup vs baseline: 10.8584x; 10.8584x over previous
import functools

import jax
import jax.numpy as jnp
from jax import lax
from jax.experimental import pallas as pl
from jax.experimental.pallas import tpu as pltpu
from jax.experimental.pallas import tpu_sc as plsc

N = 10000
E = 320000
D = 128
G = 64
NC = 2
NS = 16
NW = NC * NS
EPT = E // NW
C = 80
K = EPT // C
NP = 10240
RPT = NP // NS

f32 = jnp.float32

_MESH = plsc.VectorSubcoreMesh(core_axis_name="c", subcore_axis_name="s")


def _sc_deg_body(dst3, z128, ones, dout, dst_v, ones_v, dacc):
    c = lax.axis_index("c")
    s = lax.axis_index("s")
    wid = s * NC + c
    r0 = pl.multiple_of(s * RPT, 8)
    pltpu.sync_copy(z128.at[pl.ds(r0, RPT)], dacc.at[pl.ds(r0, RPT)])
    pltpu.sync_copy(ones, ones_v)
    pltpu.sync_copy(dst3.at[wid], dst_v)
    plsc.subcore_barrier()

    def chunk(j, carry):
        pltpu.sync_copy(ones_v, dacc.at[dst_v.at[j]], add=True)
        return carry

    lax.fori_loop(0, K, chunk, 0)
    plsc.subcore_barrier()
    pltpu.sync_copy(dacc.at[pl.ds(r0, RPT)], dout.at[c, pl.ds(r0, RPT)])


def _sc_step_body(h, src3, dst3, z128, out,
                  src_v, dst_v, rows_v, acc, sem):
    c = lax.axis_index("c")
    s = lax.axis_index("s")
    wid = s * NC + c
    r0 = pl.multiple_of(s * RPT, 8)
    pltpu.sync_copy(z128.at[pl.ds(r0, RPT)], acc.at[pl.ds(r0, RPT)])
    pltpu.sync_copy(src3.at[wid], src_v)
    pltpu.sync_copy(dst3.at[wid], dst_v)
    plsc.subcore_barrier()

    def chunk(j, carry):
        pltpu.async_copy(h.at[src_v.at[j]], rows_v, sem).wait()
        pltpu.sync_copy(rows_v, acc.at[dst_v.at[j]], add=True)
        return carry

    lax.fori_loop(0, K, chunk, 0)
    plsc.subcore_barrier()
    pltpu.sync_copy(acc.at[pl.ds(r0, RPT)], out.at[c, pl.ds(r0, RPT)])


_sc_deg = pl.kernel(
    _sc_deg_body,
    out_type=jax.ShapeDtypeStruct((NC, NP, D), f32),
    mesh=_MESH,
    scratch_types=[
        pltpu.VMEM((K, C), jnp.int32),
        pltpu.VMEM((C, D), f32),
        pltpu.VMEM_SHARED((NP, D), f32),
    ],
)

_sc_step = pl.kernel(
    _sc_step_body,
    out_type=jax.ShapeDtypeStruct((NC, NP, D), f32),
    mesh=_MESH,
    scratch_types=[
        pltpu.VMEM((K, C), jnp.int32),
        pltpu.VMEM((K, C), jnp.int32),
        pltpu.VMEM((C, D), f32),
        pltpu.VMEM_SHARED((NP, D), f32),
        pltpu.SemaphoreType.DMA,
    ],
)


def _input_body(x_ref, w_ref, b_ref, o_ref):
    y = lax.dot_general(x_ref[...], w_ref[...], (((1,), (1,)), ((), ())),
                        preferred_element_type=f32)
    o_ref[...] = jnp.maximum(y + b_ref[...], 0.0)


def _deginv_body(d_ref, o_ref):
    o_ref[...] = 1.0 / jnp.maximum(d_ref[0, :N] + d_ref[1, :N], 1.0)


def _combine_body(h_ref, p_ref, di_ref, o_ref):
    psum = p_ref[0, :N] + p_ref[1, :N]
    o_ref[...] = (h_ref[...] + psum * di_ref[...]) * 0.5


def _s2s_body(h_ref, b_ref, bt_ref, wih_ref, whh_ref, bih_ref, bhh_ref,
              wp_ref, bp_ref, o_ref):
    h = h_ref[...]
    M = (b_ref[...] == lax.broadcasted_iota(jnp.int32, (N, G), 1)).astype(f32)
    Mt = (bt_ref[...] == lax.broadcasted_iota(jnp.int32, (G, N), 0)).astype(f32)
    q_star = jnp.zeros((G, 2 * D), f32)
    hs = jnp.zeros((G, D), f32)
    cs = jnp.zeros((G, D), f32)
    for _ in range(3):
        gates = (lax.dot_general(q_star, wih_ref[...], (((1,), (1,)), ((), ())),
                                 preferred_element_type=f32)
                 + bih_ref[...]
                 + lax.dot_general(hs, whh_ref[...], (((1,), (1,)), ((), ())),
                                   preferred_element_type=f32)
                 + bhh_ref[...])
        i_g = jax.nn.sigmoid(gates[:, 0:D])
        f_g = jax.nn.sigmoid(gates[:, D:2 * D])
        g_g = jnp.tanh(gates[:, 2 * D:3 * D])
        o_g = jax.nn.sigmoid(gates[:, 3 * D:4 * D])
        cs = f_g * cs + i_g * g_g
        hs = o_g * jnp.tanh(cs)
        qb = lax.dot_general(M, hs, (((1,), (0,)), ((), ())),
                             precision=lax.Precision.HIGHEST,
                             preferred_element_type=f32)
        e = jnp.sum(h * qb, axis=1, keepdims=True)
        em = jnp.max(jnp.where(M > 0, e, -1e30), axis=0, keepdims=True)
        emn = jnp.sum(M * em, axis=1, keepdims=True)
        a = jnp.exp(e - emn)
        as_g = jnp.sum(M * a, axis=0, keepdims=True)
        asn = jnp.sum(M * as_g, axis=1, keepdims=True)
        an = a / asn
        r = lax.dot_general(Mt, an * h, (((1,), (0,)), ((), ())),
                            precision=lax.Precision.HIGHEST,
                            preferred_element_type=f32)
        q_star = jnp.concatenate([hs, r], axis=1)
    proj = lax.dot_general(q_star, wp_ref[...], (((1,), (1,)), ((), ())),
                           preferred_element_type=f32)
    o_ref[...] = proj[:, 0:1] + bp_ref[...]


_input_layer = pl.pallas_call(
    _input_body, out_shape=jax.ShapeDtypeStruct((N, D), f32))

_deginv = pl.pallas_call(
    _deginv_body, out_shape=jax.ShapeDtypeStruct((N, D), f32))

_combine = pl.pallas_call(
    _combine_body, out_shape=jax.ShapeDtypeStruct((N, D), f32))

_s2s = pl.pallas_call(
    _s2s_body, out_shape=jax.ShapeDtypeStruct((G, 1), f32))


def kernel(x, edge_index, batch, W_in, b_in, W_ih, W_hh, b_ih, b_hh,
           W_pred, b_pred):
    src3 = edge_index[0].astype(jnp.int32).reshape(NW, K, C)
    dst3 = edge_index[1].astype(jnp.int32).reshape(NW, K, C)
    z128 = jnp.zeros((NP, D), f32)
    ones = jnp.ones((C, D), f32)
    b2 = batch.astype(jnp.int32).reshape(N, 1)
    bt = batch.astype(jnp.int32).reshape(1, N)

    h = _input_layer(x, W_in, b_in.reshape(1, D))
    dinv = _deginv(_sc_deg(dst3, z128, ones))
    for _ in range(3):
        p = _sc_step(h, src3, dst3, z128)
        h = _combine(h, p, dinv)
    wpad = jnp.zeros((8, 2 * D), f32).at[0].set(W_pred[0])
    return _s2s(h, b2, bt, W_ih, W_hh, b_ih.reshape(1, 4 * D),
                b_hh.reshape(1, 4 * D), wpad, b_pred.reshape(1, 1))
